# Initial kernel scaffold; baseline (speedup 1.0000x reference)
import jax
import jax.numpy as jnp
from jax.experimental import pallas as pl


def kernel(x, edge_index, W1, b1, W2, b2, W3, b3):
    raise NotImplementedError("write your pallas kernel here")


# speedup vs baseline: 21.6019x; 21.6019x over previous
import functools

import jax
import jax.numpy as jnp
from jax import lax
from jax.experimental import pallas as pl
from jax.experimental.pallas import tpu as pltpu
from jax.experimental.pallas import tpu_sc as plsc

N = 10000
NP = 10240
E = 320000
NC = 2
NS = 16
NW = NC * NS
EPW = E // NW
K = 80
CH = EPW // K
ROWS = NP // NS

def _mesh():
    return plsc.VectorSubcoreMesh(core_axis_name="c", subcore_axis_name="s")


@functools.cache
def _make_sc_deg():
    @functools.partial(
        pl.kernel,
        mesh=_mesh(),
        compiler_params=pltpu.CompilerParams(use_tc_tiling_on_sc=False),
        out_type=jax.ShapeDtypeStruct((NC, NP, 1), jnp.float32),
        scratch_types=[
            pltpu.VMEM((CH, K), jnp.int32),
            pltpu.VMEM((K, 1), jnp.float32),
            pltpu.VMEM_SHARED((NP, 1), jnp.float32),
        ],
    )
    def k(dst_hbm, ones_hbm, zero_hbm, out_hbm, dst_v, ones_v, acc):
        c = lax.axis_index("c")
        s = lax.axis_index("s")
        wid = s * NC + c
        pltpu.sync_copy(zero_hbm.at[pl.ds(s * ROWS, ROWS)],
                        acc.at[pl.ds(s * ROWS, ROWS)])
        pltpu.sync_copy(dst_hbm.at[wid], dst_v)
        pltpu.sync_copy(ones_hbm, ones_v)
        plsc.subcore_barrier()

        def body(j, carry):
            pltpu.sync_copy(ones_v, acc.at[dst_v.at[j]], add=True)
            return carry

        lax.fori_loop(0, CH, body, 0)
        plsc.subcore_barrier()
        pltpu.sync_copy(acc.at[pl.ds(s * ROWS, ROWS)],
                        out_hbm.at[c, pl.ds(s * ROWS, ROWS)])

    return k


@functools.cache
def _make_sc_agg(D):
    @functools.partial(
        pl.kernel,
        mesh=_mesh(),
        compiler_params=pltpu.CompilerParams(use_tc_tiling_on_sc=False),
        out_type=jax.ShapeDtypeStruct((NC, NP, D), jnp.float32),
        scratch_types=[
            pltpu.VMEM((CH, K), jnp.int32),
            pltpu.VMEM((CH, K), jnp.int32),
            pltpu.VMEM((K, D), jnp.float32),
            pltpu.VMEM_SHARED((NP, D), jnp.float32),
            pltpu.SemaphoreType.DMA,
        ],
    )
    def k(g_hbm, src_hbm, dst_hbm, zero_hbm, out_hbm,
          src_v, dst_v, rows_v, acc, sem):
        c = lax.axis_index("c")
        s = lax.axis_index("s")
        wid = s * NC + c
        pltpu.sync_copy(zero_hbm.at[pl.ds(s * ROWS, ROWS)],
                        acc.at[pl.ds(s * ROWS, ROWS)])
        pltpu.sync_copy(src_hbm.at[wid], src_v)
        pltpu.sync_copy(dst_hbm.at[wid], dst_v)
        plsc.subcore_barrier()

        def body(j, carry):
            pltpu.async_copy(g_hbm.at[src_v.at[j]], rows_v, sem).wait()
            pltpu.sync_copy(rows_v, acc.at[dst_v.at[j]], add=True)
            return carry

        lax.fori_loop(0, CH, body, 0)
        plsc.subcore_barrier()
        pltpu.sync_copy(acc.at[pl.ds(s * ROWS, ROWS)],
                        out_hbm.at[c, pl.ds(s * ROWS, ROWS)])

    return k


def _sc_deg(*a):
    return _make_sc_deg()(*a)


def _sc_agg(D, *a):
    return _make_sc_agg(D)(*a)


_BLK = 2048
_GRID = NP // _BLK


def _row_spec(d):
    return pl.BlockSpec((_BLK, d), lambda i: (i, 0))


def _full_spec(r, d):
    return pl.BlockSpec((r, d), lambda i: (0, 0))


def _tc_a_body(d0, d1, x, w1, g1, dinv):
    deg = d0[...] + d1[...] + 1.0
    di = lax.rsqrt(deg)
    g1[...] = di * jnp.dot(x[...], w1[...],
                           preferred_element_type=jnp.float32)
    dinv[...] = di


@jax.jit
def _tc_a(d0, d1, x, W1):
    return pl.pallas_call(
        _tc_a_body,
        grid=(_GRID,),
        in_specs=[_row_spec(1), _row_spec(1), _row_spec(128),
                  _full_spec(128, 64)],
        out_specs=[_row_spec(64), _row_spec(1)],
        out_shape=[jax.ShapeDtypeStruct((NP, 64), jnp.float32),
                   jax.ShapeDtypeStruct((NP, 1), jnp.float32)],
    )(d0, d1, x, W1)


def _tc_mid_body(p0, p1, g, dinv, b, w, gn):
    di = dinv[...]
    t = jnp.maximum(di * (p0[...] + p1[...] + g[...]) + b[...], 0.0)
    gn[...] = di * jnp.dot(t, w[...], preferred_element_type=jnp.float32)


def _make_tc_mid(din, dout):
    @jax.jit
    def f(p0, p1, g, dinv, b, W):
        return pl.pallas_call(
            _tc_mid_body,
            grid=(_GRID,),
            in_specs=[_row_spec(din), _row_spec(din), _row_spec(din),
                      _row_spec(1), _full_spec(1, din),
                      _full_spec(din, dout)],
            out_specs=_row_spec(dout),
            out_shape=jax.ShapeDtypeStruct((NP, dout), jnp.float32),
        )(p0, p1, g, dinv, b, W)
    return f


_tc_b = _make_tc_mid(64, 32)
_tc_c = _make_tc_mid(32, 16)


def _tc_d_body(p0, p1, g, dinv, b, out):
    z = dinv[...] * (p0[...] + p1[...] + g[...]) + b[...]
    z2 = z[:, 0:2]
    m = jnp.max(z2, axis=1, keepdims=True)
    lse = m + jnp.log(jnp.sum(jnp.exp(z2 - m), axis=1, keepdims=True))
    out[...] = z2 - lse


@jax.jit
def _tc_d(p0, p1, g, dinv, b):
    return pl.pallas_call(
        _tc_d_body,
        grid=(_GRID,),
        in_specs=[_row_spec(16), _row_spec(16), _row_spec(16),
                  _row_spec(1), _full_spec(1, 16)],
        out_specs=_row_spec(2),
        out_shape=jax.ShapeDtypeStruct((NP, 2), jnp.float32),
    )(p0, p1, g, dinv, b)


def kernel(x, edge_index, W1, b1, W2, b2, W3, b3):
    src3 = edge_index[0].astype(jnp.int32).reshape(NW, CH, K)
    dst3 = edge_index[1].astype(jnp.int32).reshape(NW, CH, K)

    xp = jnp.pad(x, ((0, NP - N), (0, 0)))
    ones_k = jnp.ones((K, 1), jnp.float32)
    z1 = jnp.zeros((NP, 1), jnp.float32)
    z64 = jnp.zeros((NP, 64), jnp.float32)
    z32 = jnp.zeros((NP, 32), jnp.float32)
    z16 = jnp.zeros((NP, 16), jnp.float32)

    W3p = jnp.pad(W3, ((0, 0), (0, 14)))
    b1r = b1.reshape(1, 64)
    b2r = b2.reshape(1, 32)
    b3p = jnp.pad(b3, (0, 14)).reshape(1, 16)

    degp = _sc_deg(dst3, ones_k, z1)
    g1, dinv = _tc_a(degp[0], degp[1], xp, W1)

    p = _sc_agg(64, g1, src3, dst3, z64)
    g2 = _tc_b(p[0], p[1], g1, dinv, b1r, W2)

    q = _sc_agg(32, g2, src3, dst3, z32)
    g3 = _tc_c(q[0], q[1], g2, dinv, b2r, W3p)

    r = _sc_agg(16, g3, src3, dst3, z16)
    return _tc_d(r[0], r[1], g3, dinv, b3p)[:N]


# speedup vs baseline: 33.6676x; 1.5585x over previous
import functools

import jax
import jax.numpy as jnp
from jax import lax
from jax.experimental import pallas as pl
from jax.experimental.pallas import tpu as pltpu
from jax.experimental.pallas import tpu_sc as plsc

N = 10000
NP = 10240
E = 320000
NC = 2
NS = 16
NW = NC * NS
EPW = E // NW
K = 80
CH = EPW // K
ROWS = NP // NS

def _mesh():
    return plsc.VectorSubcoreMesh(core_axis_name="c", subcore_axis_name="s")


NB = 5
NGRP = CH // NB


@functools.cache
def _make_sc_deg():
    @functools.partial(
        pl.kernel,
        mesh=_mesh(),
        compiler_params=pltpu.CompilerParams(use_tc_tiling_on_sc=False),
        out_type=jax.ShapeDtypeStruct((NC, NP, 1), jnp.float32),
        scratch_types=[
            pltpu.VMEM((CH, K), jnp.int32),
            pltpu.VMEM((K, 1), jnp.float32),
            pltpu.VMEM_SHARED((NP, 1), jnp.float32),
            pltpu.SemaphoreType.DMA,
        ],
    )
    def k(dst_hbm, ones_hbm, zero_hbm, out_hbm, dst_v, ones_v, acc, ssem):
        c = lax.axis_index("c")
        s = lax.axis_index("s")
        wid = s * NC + c
        pltpu.sync_copy(zero_hbm.at[pl.ds(s * ROWS, ROWS)],
                        acc.at[pl.ds(s * ROWS, ROWS)])
        pltpu.sync_copy(dst_hbm.at[wid], dst_v)
        pltpu.sync_copy(ones_hbm, ones_v)
        plsc.subcore_barrier()

        def body(g, carry):
            ds = [pltpu.async_copy(ones_v, acc.at[dst_v.at[g * NB + b]],
                                   ssem, add=True) for b in range(NB)]
            for d in ds:
                d.wait()
            return carry

        lax.fori_loop(0, NGRP, body, 0)
        plsc.subcore_barrier()
        pltpu.sync_copy(acc.at[pl.ds(s * ROWS, ROWS)],
                        out_hbm.at[c, pl.ds(s * ROWS, ROWS)])

    return k


@functools.cache
def _make_sc_agg(D):
    @functools.partial(
        pl.kernel,
        mesh=_mesh(),
        compiler_params=pltpu.CompilerParams(use_tc_tiling_on_sc=False),
        out_type=jax.ShapeDtypeStruct((NC, NP, D), jnp.float32),
        scratch_types=[
            pltpu.VMEM((CH, K), jnp.int32),
            pltpu.VMEM((CH, K), jnp.int32),
            pltpu.VMEM((NB, K, D), jnp.float32),
            pltpu.VMEM_SHARED((NP, D), jnp.float32),
            pltpu.SemaphoreType.DMA,
            pltpu.SemaphoreType.DMA,
        ],
    )
    def k(g_hbm, src_hbm, dst_hbm, zero_hbm, out_hbm,
          src_v, dst_v, bufs, acc, gsem, ssem):
        c = lax.axis_index("c")
        s = lax.axis_index("s")
        wid = s * NC + c
        pltpu.sync_copy(zero_hbm.at[pl.ds(s * ROWS, ROWS)],
                        acc.at[pl.ds(s * ROWS, ROWS)])
        pltpu.sync_copy(src_hbm.at[wid], src_v)
        pltpu.sync_copy(dst_hbm.at[wid], dst_v)
        plsc.subcore_barrier()

        def body(g, carry):
            gs = [pltpu.async_copy(g_hbm.at[src_v.at[g * NB + b]],
                                   bufs.at[b], gsem) for b in range(NB)]
            for d in gs:
                d.wait()
            ss = [pltpu.async_copy(bufs.at[b],
                                   acc.at[dst_v.at[g * NB + b]],
                                   ssem, add=True) for b in range(NB)]
            for d in ss:
                d.wait()
            return carry

        lax.fori_loop(0, NGRP, body, 0)
        plsc.subcore_barrier()
        pltpu.sync_copy(acc.at[pl.ds(s * ROWS, ROWS)],
                        out_hbm.at[c, pl.ds(s * ROWS, ROWS)])

    return k


def _sc_deg(*a):
    return _make_sc_deg()(*a)


def _sc_agg(D, *a):
    return _make_sc_agg(D)(*a)


_BLK = 2048
_GRID = NP // _BLK


def _row_spec(d):
    return pl.BlockSpec((_BLK, d), lambda i: (i, 0))


def _full_spec(r, d):
    return pl.BlockSpec((r, d), lambda i: (0, 0))


def _tc_a_body(d0, d1, x, w1, g1, dinv):
    deg = d0[...] + d1[...] + 1.0
    di = lax.rsqrt(deg)
    g1[...] = di * jnp.dot(x[...], w1[...],
                           preferred_element_type=jnp.float32)
    dinv[...] = di


@jax.jit
def _tc_a(d0, d1, x, W1):
    return pl.pallas_call(
        _tc_a_body,
        grid=(_GRID,),
        in_specs=[_row_spec(1), _row_spec(1), _row_spec(128),
                  _full_spec(128, 64)],
        out_specs=[_row_spec(64), _row_spec(1)],
        out_shape=[jax.ShapeDtypeStruct((NP, 64), jnp.float32),
                   jax.ShapeDtypeStruct((NP, 1), jnp.float32)],
    )(d0, d1, x, W1)


def _tc_mid_body(p0, p1, g, dinv, b, w, gn):
    di = dinv[...]
    t = jnp.maximum(di * (p0[...] + p1[...] + g[...]) + b[...], 0.0)
    gn[...] = di * jnp.dot(t, w[...], preferred_element_type=jnp.float32)


def _make_tc_mid(din, dout):
    @jax.jit
    def f(p0, p1, g, dinv, b, W):
        return pl.pallas_call(
            _tc_mid_body,
            grid=(_GRID,),
            in_specs=[_row_spec(din), _row_spec(din), _row_spec(din),
                      _row_spec(1), _full_spec(1, din),
                      _full_spec(din, dout)],
            out_specs=_row_spec(dout),
            out_shape=jax.ShapeDtypeStruct((NP, dout), jnp.float32),
        )(p0, p1, g, dinv, b, W)
    return f


_tc_b = _make_tc_mid(64, 32)
_tc_c = _make_tc_mid(32, 16)


def _tc_d_body(p0, p1, g, dinv, b, out):
    z = dinv[...] * (p0[...] + p1[...] + g[...]) + b[...]
    z2 = z[:, 0:2]
    m = jnp.max(z2, axis=1, keepdims=True)
    lse = m + jnp.log(jnp.sum(jnp.exp(z2 - m), axis=1, keepdims=True))
    out[...] = z2 - lse


@jax.jit
def _tc_d(p0, p1, g, dinv, b):
    return pl.pallas_call(
        _tc_d_body,
        grid=(_GRID,),
        in_specs=[_row_spec(16), _row_spec(16), _row_spec(16),
                  _row_spec(1), _full_spec(1, 16)],
        out_specs=_row_spec(2),
        out_shape=jax.ShapeDtypeStruct((NP, 2), jnp.float32),
    )(p0, p1, g, dinv, b)


def kernel(x, edge_index, W1, b1, W2, b2, W3, b3):
    src3 = edge_index[0].astype(jnp.int32).reshape(NW, CH, K)
    dst3 = edge_index[1].astype(jnp.int32).reshape(NW, CH, K)

    xp = jnp.pad(x, ((0, NP - N), (0, 0)))
    ones_k = jnp.ones((K, 1), jnp.float32)
    z1 = jnp.zeros((NP, 1), jnp.float32)
    z64 = jnp.zeros((NP, 64), jnp.float32)
    z32 = jnp.zeros((NP, 32), jnp.float32)
    z16 = jnp.zeros((NP, 16), jnp.float32)

    W3p = jnp.pad(W3, ((0, 0), (0, 14)))
    b1r = b1.reshape(1, 64)
    b2r = b2.reshape(1, 32)
    b3p = jnp.pad(b3, (0, 14)).reshape(1, 16)

    degp = _sc_deg(dst3, ones_k, z1)
    g1, dinv = _tc_a(degp[0], degp[1], xp, W1)

    p = _sc_agg(64, g1, src3, dst3, z64)
    g2 = _tc_b(p[0], p[1], g1, dinv, b1r, W2)

    q = _sc_agg(32, g2, src3, dst3, z32)
    g3 = _tc_c(q[0], q[1], g2, dinv, b2r, W3p)

    r = _sc_agg(16, g3, src3, dst3, z16)
    return _tc_d(r[0], r[1], g3, dinv, b3p)[:N]


# speedup vs baseline: 35.5866x; 1.0570x over previous
import functools

import jax
import jax.numpy as jnp
from jax import lax
from jax.experimental import pallas as pl
from jax.experimental.pallas import tpu as pltpu
from jax.experimental.pallas import tpu_sc as plsc

N = 10000
NP = 10240
E = 320000
NC = 2
NS = 16
NW = NC * NS
EPW = E // NW
K = 80
CH = EPW // K
ROWS = NP // NS

def _mesh():
    return plsc.VectorSubcoreMesh(core_axis_name="c", subcore_axis_name="s")


NB = 10
NGRP = CH // NB
TAIL = CH - NGRP * NB


@functools.cache
def _make_sc_deg():
    @functools.partial(
        pl.kernel,
        mesh=_mesh(),
        compiler_params=pltpu.CompilerParams(use_tc_tiling_on_sc=False),
        out_type=jax.ShapeDtypeStruct((NC, NP, 1), jnp.float32),
        scratch_types=[
            pltpu.VMEM((CH, K), jnp.int32),
            pltpu.VMEM((K, 1), jnp.float32),
            pltpu.VMEM_SHARED((NP, 1), jnp.float32),
            pltpu.SemaphoreType.DMA,
        ],
    )
    def k(dst_hbm, ones_hbm, zero_hbm, out_hbm, dst_v, ones_v, acc, ssem):
        c = lax.axis_index("c")
        s = lax.axis_index("s")
        wid = s * NC + c
        pltpu.sync_copy(zero_hbm.at[pl.ds(s * ROWS, ROWS)],
                        acc.at[pl.ds(s * ROWS, ROWS)])
        pltpu.sync_copy(dst_hbm.at[wid], dst_v)
        pltpu.sync_copy(ones_hbm, ones_v)
        plsc.subcore_barrier()

        def run_grp(base, n):
            ds = [pltpu.async_copy(ones_v, acc.at[dst_v.at[base + b]],
                                   ssem, add=True) for b in range(n)]
            for d in ds:
                d.wait()

        def body(g, carry):
            run_grp(g * NB, NB)
            return carry

        lax.fori_loop(0, NGRP, body, 0)
        run_grp(NGRP * NB, TAIL)
        plsc.subcore_barrier()
        pltpu.sync_copy(acc.at[pl.ds(s * ROWS, ROWS)],
                        out_hbm.at[c, pl.ds(s * ROWS, ROWS)])

    return k


@functools.cache
def _make_sc_agg(D):
    @functools.partial(
        pl.kernel,
        mesh=_mesh(),
        compiler_params=pltpu.CompilerParams(use_tc_tiling_on_sc=False),
        out_type=jax.ShapeDtypeStruct((NC, NP, D), jnp.float32),
        scratch_types=[
            pltpu.VMEM((CH, K), jnp.int32),
            pltpu.VMEM((CH, K), jnp.int32),
            pltpu.VMEM((NB, K, D), jnp.float32),
            pltpu.VMEM_SHARED((NP, D), jnp.float32),
            pltpu.SemaphoreType.DMA,
            pltpu.SemaphoreType.DMA,
        ],
    )
    def k(g_hbm, src_hbm, dst_hbm, zero_hbm, out_hbm,
          src_v, dst_v, bufs, acc, gsem, ssem):
        c = lax.axis_index("c")
        s = lax.axis_index("s")
        wid = s * NC + c
        pltpu.sync_copy(zero_hbm.at[pl.ds(s * ROWS, ROWS)],
                        acc.at[pl.ds(s * ROWS, ROWS)])
        pltpu.sync_copy(src_hbm.at[wid], src_v)
        pltpu.sync_copy(dst_hbm.at[wid], dst_v)
        plsc.subcore_barrier()

        def run_grp(base, n):
            gs = [pltpu.async_copy(g_hbm.at[src_v.at[base + b]],
                                   bufs.at[b], gsem) for b in range(n)]
            for d in gs:
                d.wait()
            ss = [pltpu.async_copy(bufs.at[b],
                                   acc.at[dst_v.at[base + b]],
                                   ssem, add=True) for b in range(n)]
            for d in ss:
                d.wait()

        def body(g, carry):
            run_grp(g * NB, NB)
            return carry

        lax.fori_loop(0, NGRP, body, 0)
        run_grp(NGRP * NB, TAIL)
        plsc.subcore_barrier()
        pltpu.sync_copy(acc.at[pl.ds(s * ROWS, ROWS)],
                        out_hbm.at[c, pl.ds(s * ROWS, ROWS)])

    return k


def _sc_deg(*a):
    return _make_sc_deg()(*a)


def _sc_agg(D, *a):
    return _make_sc_agg(D)(*a)


_BLK = 2048
_GRID = NP // _BLK


def _row_spec(d):
    return pl.BlockSpec((_BLK, d), lambda i: (i, 0))


def _full_spec(r, d):
    return pl.BlockSpec((r, d), lambda i: (0, 0))


def _tc_a_body(d0, d1, x, w1, g1, dinv):
    deg = d0[...] + d1[...] + 1.0
    di = lax.rsqrt(deg)
    g1[...] = di * jnp.dot(x[...], w1[...],
                           preferred_element_type=jnp.float32)
    dinv[...] = di


@jax.jit
def _tc_a(d0, d1, x, W1):
    return pl.pallas_call(
        _tc_a_body,
        grid=(_GRID,),
        in_specs=[_row_spec(1), _row_spec(1), _row_spec(128),
                  _full_spec(128, 64)],
        out_specs=[_row_spec(64), _row_spec(1)],
        out_shape=[jax.ShapeDtypeStruct((NP, 64), jnp.float32),
                   jax.ShapeDtypeStruct((NP, 1), jnp.float32)],
    )(d0, d1, x, W1)


def _tc_mid_body(p0, p1, g, dinv, b, w, gn):
    di = dinv[...]
    t = jnp.maximum(di * (p0[...] + p1[...] + g[...]) + b[...], 0.0)
    gn[...] = di * jnp.dot(t, w[...], preferred_element_type=jnp.float32)


def _make_tc_mid(din, dout):
    @jax.jit
    def f(p0, p1, g, dinv, b, W):
        return pl.pallas_call(
            _tc_mid_body,
            grid=(_GRID,),
            in_specs=[_row_spec(din), _row_spec(din), _row_spec(din),
                      _row_spec(1), _full_spec(1, din),
                      _full_spec(din, dout)],
            out_specs=_row_spec(dout),
            out_shape=jax.ShapeDtypeStruct((NP, dout), jnp.float32),
        )(p0, p1, g, dinv, b, W)
    return f


_tc_b = _make_tc_mid(64, 32)
_tc_c = _make_tc_mid(32, 16)


def _tc_d_body(p0, p1, g, dinv, b, out):
    z = dinv[...] * (p0[...] + p1[...] + g[...]) + b[...]
    z2 = z[:, 0:2]
    m = jnp.max(z2, axis=1, keepdims=True)
    lse = m + jnp.log(jnp.sum(jnp.exp(z2 - m), axis=1, keepdims=True))
    out[...] = z2 - lse


@jax.jit
def _tc_d(p0, p1, g, dinv, b):
    return pl.pallas_call(
        _tc_d_body,
        grid=(_GRID,),
        in_specs=[_row_spec(16), _row_spec(16), _row_spec(16),
                  _row_spec(1), _full_spec(1, 16)],
        out_specs=_row_spec(2),
        out_shape=jax.ShapeDtypeStruct((NP, 2), jnp.float32),
    )(p0, p1, g, dinv, b)


def kernel(x, edge_index, W1, b1, W2, b2, W3, b3):
    src3 = edge_index[0].astype(jnp.int32).reshape(NW, CH, K)
    dst3 = edge_index[1].astype(jnp.int32).reshape(NW, CH, K)

    xp = jnp.pad(x, ((0, NP - N), (0, 0)))
    ones_k = jnp.ones((K, 1), jnp.float32)
    z1 = jnp.zeros((NP, 1), jnp.float32)
    z64 = jnp.zeros((NP, 64), jnp.float32)
    z32 = jnp.zeros((NP, 32), jnp.float32)
    z16 = jnp.zeros((NP, 16), jnp.float32)

    W3p = jnp.pad(W3, ((0, 0), (0, 14)))
    b1r = b1.reshape(1, 64)
    b2r = b2.reshape(1, 32)
    b3p = jnp.pad(b3, (0, 14)).reshape(1, 16)

    degp = _sc_deg(dst3, ones_k, z1)
    g1, dinv = _tc_a(degp[0], degp[1], xp, W1)

    p = _sc_agg(64, g1, src3, dst3, z64)
    g2 = _tc_b(p[0], p[1], g1, dinv, b1r, W2)

    q = _sc_agg(32, g2, src3, dst3, z32)
    g3 = _tc_c(q[0], q[1], g2, dinv, b2r, W3p)

    r = _sc_agg(16, g3, src3, dst3, z16)
    return _tc_d(r[0], r[1], g3, dinv, b3p)[:N]


# speedup vs baseline: 36.1050x; 1.0146x over previous
import functools

import jax
import jax.numpy as jnp
from jax import lax
from jax.experimental import pallas as pl
from jax.experimental.pallas import tpu as pltpu
from jax.experimental.pallas import tpu_sc as plsc

N = 10000
NP = 10240
E = 320000
NC = 2
NS = 16
NW = NC * NS
EPW = E // NW
K = 80
CH = EPW // K
ROWS = NP // NS

def _mesh():
    return plsc.VectorSubcoreMesh(core_axis_name="c", subcore_axis_name="s")


def _nb_for(width):
    return 12 if width >= 64 else 15


@functools.cache
def _make_sc_deg():
    @functools.partial(
        pl.kernel,
        mesh=_mesh(),
        compiler_params=pltpu.CompilerParams(use_tc_tiling_on_sc=False),
        out_type=jax.ShapeDtypeStruct((NC, NP, 1), jnp.float32),
        scratch_types=[
            pltpu.VMEM((CH, K), jnp.int32),
            pltpu.VMEM((K, 1), jnp.float32),
            pltpu.VMEM_SHARED((NP, 1), jnp.float32),
            pltpu.SemaphoreType.DMA,
        ],
    )
    def k(dst_hbm, ones_hbm, zero_hbm, out_hbm, dst_v, ones_v, acc, ssem):
        c = lax.axis_index("c")
        s = lax.axis_index("s")
        wid = s * NC + c
        pltpu.sync_copy(zero_hbm.at[pl.ds(s * ROWS, ROWS)],
                        acc.at[pl.ds(s * ROWS, ROWS)])
        pltpu.sync_copy(dst_hbm.at[wid], dst_v)
        pltpu.sync_copy(ones_hbm, ones_v)
        plsc.subcore_barrier()

        nb = _nb_for(1)
        ngrp, tail = CH // nb, CH % nb

        def run_grp(base, n):
            ds = [pltpu.async_copy(ones_v, acc.at[dst_v.at[base + b]],
                                   ssem, add=True) for b in range(n)]
            for d in ds:
                d.wait()

        def body(g, carry):
            run_grp(g * nb, nb)
            return carry

        lax.fori_loop(0, ngrp, body, 0)
        run_grp(ngrp * nb, tail)
        plsc.subcore_barrier()
        pltpu.sync_copy(acc.at[pl.ds(s * ROWS, ROWS)],
                        out_hbm.at[c, pl.ds(s * ROWS, ROWS)])

    return k


@functools.cache
def _make_sc_agg(D):
    @functools.partial(
        pl.kernel,
        mesh=_mesh(),
        compiler_params=pltpu.CompilerParams(use_tc_tiling_on_sc=False),
        out_type=jax.ShapeDtypeStruct((NC, NP, D), jnp.float32),
        scratch_types=[
            pltpu.VMEM((CH, K), jnp.int32),
            pltpu.VMEM((CH, K), jnp.int32),
            pltpu.VMEM((_nb_for(D), K, D), jnp.float32),
            pltpu.VMEM_SHARED((NP, D), jnp.float32),
            pltpu.SemaphoreType.DMA,
            pltpu.SemaphoreType.DMA,
        ],
    )
    def k(g_hbm, src_hbm, dst_hbm, zero_hbm, out_hbm,
          src_v, dst_v, bufs, acc, gsem, ssem):
        c = lax.axis_index("c")
        s = lax.axis_index("s")
        wid = s * NC + c
        pltpu.sync_copy(zero_hbm.at[pl.ds(s * ROWS, ROWS)],
                        acc.at[pl.ds(s * ROWS, ROWS)])
        pltpu.sync_copy(src_hbm.at[wid], src_v)
        pltpu.sync_copy(dst_hbm.at[wid], dst_v)
        plsc.subcore_barrier()

        nb = _nb_for(D)
        ngrp, tail = CH // nb, CH % nb

        def run_grp(base, n):
            gs = [pltpu.async_copy(g_hbm.at[src_v.at[base + b]],
                                   bufs.at[b], gsem) for b in range(n)]
            for d in gs:
                d.wait()
            ss = [pltpu.async_copy(bufs.at[b],
                                   acc.at[dst_v.at[base + b]],
                                   ssem, add=True) for b in range(n)]
            for d in ss:
                d.wait()

        def body(g, carry):
            run_grp(g * nb, nb)
            return carry

        lax.fori_loop(0, ngrp, body, 0)
        run_grp(ngrp * nb, tail)
        plsc.subcore_barrier()
        pltpu.sync_copy(acc.at[pl.ds(s * ROWS, ROWS)],
                        out_hbm.at[c, pl.ds(s * ROWS, ROWS)])

    return k


def _sc_deg(*a):
    return _make_sc_deg()(*a)


def _sc_agg(D, *a):
    return _make_sc_agg(D)(*a)


_BLK = 2048
_GRID = NP // _BLK


def _row_spec(d):
    return pl.BlockSpec((_BLK, d), lambda i: (i, 0))


def _full_spec(r, d):
    return pl.BlockSpec((r, d), lambda i: (0, 0))


def _tc_a_body(d0, d1, x, w1, g1, dinv):
    deg = d0[...] + d1[...] + 1.0
    di = lax.rsqrt(deg)
    g1[...] = di * jnp.dot(x[...], w1[...],
                           preferred_element_type=jnp.float32)
    dinv[...] = di


@jax.jit
def _tc_a(d0, d1, x, W1):
    return pl.pallas_call(
        _tc_a_body,
        grid=(_GRID,),
        in_specs=[_row_spec(1), _row_spec(1), _row_spec(128),
                  _full_spec(128, 64)],
        out_specs=[_row_spec(64), _row_spec(1)],
        out_shape=[jax.ShapeDtypeStruct((NP, 64), jnp.float32),
                   jax.ShapeDtypeStruct((NP, 1), jnp.float32)],
    )(d0, d1, x, W1)


def _tc_mid_body(p0, p1, g, dinv, b, w, gn):
    di = dinv[...]
    t = jnp.maximum(di * (p0[...] + p1[...] + g[...]) + b[...], 0.0)
    gn[...] = di * jnp.dot(t, w[...], preferred_element_type=jnp.float32)


def _make_tc_mid(din, dout):
    @jax.jit
    def f(p0, p1, g, dinv, b, W):
        return pl.pallas_call(
            _tc_mid_body,
            grid=(_GRID,),
            in_specs=[_row_spec(din), _row_spec(din), _row_spec(din),
                      _row_spec(1), _full_spec(1, din),
                      _full_spec(din, dout)],
            out_specs=_row_spec(dout),
            out_shape=jax.ShapeDtypeStruct((NP, dout), jnp.float32),
        )(p0, p1, g, dinv, b, W)
    return f


_tc_b = _make_tc_mid(64, 32)
_tc_c = _make_tc_mid(32, 16)


def _tc_d_body(p0, p1, g, dinv, b, out):
    z = dinv[...] * (p0[...] + p1[...] + g[...]) + b[...]
    z2 = z[:, 0:2]
    m = jnp.max(z2, axis=1, keepdims=True)
    lse = m + jnp.log(jnp.sum(jnp.exp(z2 - m), axis=1, keepdims=True))
    out[...] = z2 - lse


@jax.jit
def _tc_d(p0, p1, g, dinv, b):
    return pl.pallas_call(
        _tc_d_body,
        grid=(_GRID,),
        in_specs=[_row_spec(16), _row_spec(16), _row_spec(16),
                  _row_spec(1), _full_spec(1, 16)],
        out_specs=_row_spec(2),
        out_shape=jax.ShapeDtypeStruct((NP, 2), jnp.float32),
    )(p0, p1, g, dinv, b)


def kernel(x, edge_index, W1, b1, W2, b2, W3, b3):
    src3 = edge_index[0].astype(jnp.int32).reshape(NW, CH, K)
    dst3 = edge_index[1].astype(jnp.int32).reshape(NW, CH, K)

    xp = jnp.pad(x, ((0, NP - N), (0, 0)))
    ones_k = jnp.ones((K, 1), jnp.float32)
    z1 = jnp.zeros((NP, 1), jnp.float32)
    z64 = jnp.zeros((NP, 64), jnp.float32)
    z32 = jnp.zeros((NP, 32), jnp.float32)
    z16 = jnp.zeros((NP, 16), jnp.float32)

    W3p = jnp.pad(W3, ((0, 0), (0, 14)))
    b1r = b1.reshape(1, 64)
    b2r = b2.reshape(1, 32)
    b3p = jnp.pad(b3, (0, 14)).reshape(1, 16)

    degp = _sc_deg(dst3, ones_k, z1)
    g1, dinv = _tc_a(degp[0], degp[1], xp, W1)

    p = _sc_agg(64, g1, src3, dst3, z64)
    g2 = _tc_b(p[0], p[1], g1, dinv, b1r, W2)

    q = _sc_agg(32, g2, src3, dst3, z32)
    g3 = _tc_c(q[0], q[1], g2, dinv, b2r, W3p)

    r = _sc_agg(16, g3, src3, dst3, z16)
    return _tc_d(r[0], r[1], g3, dinv, b3p)[:N]
